# Initial kernel scaffold; baseline (speedup 1.0000x reference)
import jax
import jax.numpy as jnp
from jax.experimental import pallas as pl


def kernel(Xi, Xv, fo_tables, so_tables, W1, b1, gamma1, beta1, W2, b2, gamma2, beta2, bias):
    raise NotImplementedError("write your pallas kernel here")


# speedup vs baseline: 1.4064x; 1.4064x over previous
import functools

import jax
import jax.numpy as jnp
import numpy as np
from jax import lax
from jax.experimental import pallas as pl
from jax.experimental.pallas import tpu as pltpu
from jax.experimental.pallas import tpu_sc as plsc

B = 16384
V = 100000
D = 64
NCOLS = 45
SECTIONS = [(0, 1), (1, 2), (2, 3), (3, 4), (4, 5), (5, 6), (6, 7), (7, 10),
            (10, 24), (26, 28), (28, 30), (24, 26), (30, 45)]
BN_EPS = 1e-5

NC, NS, L = 2, 16, 16
NW = NC * NS
BPW = B // NW
NB = 256
NCHUNK = BPW // NB
D4 = D // L

_COLS = np.array([j for (s, e) in SECTIONS for j in range(s, e)], np.int32)
_OFFS = np.array([i * V for i, (s, e) in enumerate(SECTIONS)
                  for _ in range(s, e)], np.int32)

_mesh = plsc.VectorSubcoreMesh(core_axis_name="c", subcore_axis_name="s",
                               num_cores=NC, num_subcores=NS)


@functools.partial(
    pl.kernel,
    out_type=jax.ShapeDtypeStruct((B,), jnp.float32),
    mesh=_mesh,
    scratch_types=[
        pltpu.VMEM((NB,), jnp.int32),
        pltpu.VMEM((NB,), jnp.float32),
        pltpu.VMEM((NB,), jnp.float32),
        pltpu.VMEM((NB, D), jnp.float32),
        pltpu.VMEM((NB * D,), jnp.float32),
        pltpu.VMEM((NB * L,), jnp.float32),
        pltpu.VMEM((NB,), jnp.float32),
        pltpu.VMEM((NCOLS * D,), jnp.float32),
        pltpu.VMEM((L,), jnp.float32),
        pltpu.VMEM((NB,), jnp.float32),
        pltpu.SemaphoreType.DMA,
        pltpu.SemaphoreType.DMA,
    ],
    compiler_params=pltpu.CompilerParams(needs_layout_passes=False,
                                         use_tc_tiling_on_sc=False),
)
def _deepfm_sc(so_hbm, fo_hbm, idx_hbm, xv_hbm, w_hbm, cst_hbm, out_hbm,
               idx_v, xv_v, fov_v, rows_v, S_v, C_v, F_v, w_v, cst_v, out_v,
               sem_r, sem_f):
    wid = lax.axis_index("s") * NC + lax.axis_index("c")
    pltpu.sync_copy(w_hbm, w_v)
    pltpu.sync_copy(cst_hbm, cst_v)
    iota = lax.iota(jnp.int32, L)
    zero16 = jnp.zeros((L,), jnp.float32)
    zero16i = jnp.zeros((L,), jnp.int32)

    def chunk_body(c, _):
        base = wid * BPW + c * NB

        def zs_body(t, _):
            S_v[pl.ds(t * L, L)] = zero16
            return 0
        lax.fori_loop(0, NB * D // L, zs_body, 0)

        def zc_body(t, _):
            C_v[pl.ds(t * L, L)] = zero16
            return 0
        lax.fori_loop(0, NB * L // L, zc_body, 0)

        def zf_body(t, _):
            F_v[pl.ds(t * L, L)] = zero16
            return 0
        lax.fori_loop(0, NB // L, zf_body, 0)

        def col_body(p, _):
            pltpu.sync_copy(idx_hbm.at[p, pl.ds(base, NB)], idx_v)
            pltpu.sync_copy(xv_hbm.at[p, pl.ds(base, NB)], xv_v)
            cp_r = pltpu.async_copy(so_hbm.at[idx_v], rows_v, sem_r)
            cp_f = pltpu.async_copy(fo_hbm.at[idx_v], fov_v, sem_f)
            cp_r.wait()
            cp_f.wait()

            def fo_body(t, _):
                fo16 = fov_v[pl.ds(t * L, L)]
                v16 = xv_v[pl.ds(t * L, L)]
                plsc.addupdate(F_v.at[pl.ds(t * L, L)], fo16 * v16)
                return 0
            lax.fori_loop(0, NB // L, fo_body, 0)

            w0 = w_v[pl.ds(p * D + 0 * L, L)]
            w1 = w_v[pl.ds(p * D + 1 * L, L)]
            w2 = w_v[pl.ds(p * D + 2 * L, L)]
            w3 = w_v[pl.ds(p * D + 3 * L, L)]

            def b_body(b, carry):
                cw0, cw1, cw2, cw3 = carry
                vb = plsc.load_gather(xv_v, [jnp.full((L,), b, jnp.int32)])
                cacc = zero16
                for d4, wd in zip(range(D4), (cw0, cw1, cw2, cw3)):
                    r = rows_v[b, pl.ds(d4 * L, L)]
                    e = r * vb
                    plsc.addupdate(S_v.at[pl.ds(b * D + d4 * L, L)], e)
                    cacc = cacc + e * (wd - 0.5 * e)
                plsc.addupdate(C_v.at[pl.ds(b * L, L)], cacc)
                return (cw0, cw1, cw2, cw3)
            lax.fori_loop(0, NB, b_body, (w0, w1, w2, w3))
            return 0
        lax.fori_loop(0, NCOLS, col_body, 0)

        def fin_body(t, _):
            l16 = t * L + iota
            a = zero16
            for d in range(D):
                sd = plsc.load_gather(S_v, [l16 * D + d])
                a = a + sd * sd
            csum = zero16
            for k in range(L):
                ck = plsc.load_gather(C_v, [l16 * L + k])
                csum = csum + ck
            f16 = F_v[pl.ds(t * L, L)]
            out_v[pl.ds(t * L, L)] = f16 + 0.5 * a + csum + cst_v[...]
            return 0
        lax.fori_loop(0, NB // L, fin_body, 0)
        pltpu.sync_copy(out_v, out_hbm.at[pl.ds(base, NB)])
        return 0
    lax.fori_loop(0, NCHUNK, chunk_body, 0)


def kernel(Xi, Xv, fo_tables, so_tables, W1, b1, gamma1, beta1,
           W2, b2, gamma2, beta2, bias):
    a1 = 1.0 / jnp.sqrt(jnp.float32(1.0) + jnp.float32(BN_EPS))
    g2 = a1 * gamma2
    u = W2 @ g2
    g = a1 * gamma1 * u
    w_fold = W1 @ g
    cst = b1 @ g + beta1 @ u + b2 @ g2 + jnp.sum(beta2) + bias[0]
    cst_vec = jnp.full((L,), cst, jnp.float32)

    idx_pb = Xi[:, _COLS].T.astype(jnp.int32) + _OFFS[:, None]
    xv_pb = Xv[:, _COLS].T
    so_flat = so_tables.reshape(len(SECTIONS) * V, D)
    fo_flat = fo_tables.reshape(len(SECTIONS) * V)
    return _deepfm_sc(so_flat, fo_flat, idx_pb, xv_pb, w_fold, cst_vec)


# speedup vs baseline: 1.7524x; 1.2460x over previous
import functools

import jax
import jax.numpy as jnp
import numpy as np
from jax import lax
from jax.experimental import pallas as pl
from jax.experimental.pallas import tpu as pltpu
from jax.experimental.pallas import tpu_sc as plsc

B = 16384
V = 100000
D = 64
NCOLS = 45
SECTIONS = [(0, 1), (1, 2), (2, 3), (3, 4), (4, 5), (5, 6), (6, 7), (7, 10),
            (10, 24), (26, 28), (28, 30), (24, 26), (30, 45)]
BN_EPS = 1e-5

NC, NS, L = 2, 16, 16
NW = NC * NS
BPW = B // NW
NB = 256
NCHUNK = BPW // NB
D4 = D // L

_COLS = np.array([j for (s, e) in SECTIONS for j in range(s, e)], np.int32)
_OFFS_BY_COL = np.zeros((NCOLS,), np.int32)
for _i, (_s, _e) in enumerate(SECTIONS):
    _OFFS_BY_COL[_s:_e] = _i * V

_mesh = plsc.VectorSubcoreMesh(core_axis_name="c", subcore_axis_name="s",
                               num_cores=NC, num_subcores=NS)


@functools.partial(
    pl.kernel,
    out_type=jax.ShapeDtypeStruct((B,), jnp.float32),
    mesh=_mesh,
    scratch_types=[
        pltpu.VMEM((NB, NCOLS), jnp.int32),
        pltpu.VMEM((NB, NCOLS), jnp.float32),
        pltpu.VMEM((NB,), jnp.int32),
        pltpu.VMEM((NB,), jnp.int32),
        pltpu.VMEM((NB, D), jnp.float32),
        pltpu.VMEM((NB, D), jnp.float32),
        pltpu.VMEM((NB,), jnp.float32),
        pltpu.VMEM((NB,), jnp.float32),
        pltpu.VMEM((NB * D,), jnp.float32),
        pltpu.VMEM((NB * L,), jnp.float32),
        pltpu.VMEM((NB,), jnp.float32),
        pltpu.VMEM((NCOLS * D,), jnp.float32),
        pltpu.VMEM((NCOLS,), jnp.int32),
        pltpu.VMEM((L,), jnp.float32),
        pltpu.VMEM((NB,), jnp.float32),
        pltpu.SemaphoreType.DMA,
        pltpu.SemaphoreType.DMA,
        pltpu.SemaphoreType.DMA,
        pltpu.SemaphoreType.DMA,
    ],
    compiler_params=pltpu.CompilerParams(needs_layout_passes=False,
                                         use_tc_tiling_on_sc=False),
)
def _deepfm_sc(xi_hbm, xv_hbm, so_hbm, fo_hbm, w_hbm, offs_hbm, cst_hbm,
               out_hbm,
               xi_c, xv_c, idx_a, idx_b, rows_a, rows_b, fov_a, fov_b,
               S_v, C_v, F_v, w_v, offs_v, cst_v, out_v,
               sem_ra, sem_rb, sem_fa, sem_fb):
    wid = lax.axis_index("s") * NC + lax.axis_index("c")
    pltpu.sync_copy(w_hbm, w_v)
    pltpu.sync_copy(offs_hbm, offs_v)
    pltpu.sync_copy(cst_hbm, cst_v)
    iota = lax.iota(jnp.int32, L)
    zero16 = jnp.zeros((L,), jnp.float32)

    bufs = ((idx_a, rows_a, fov_a, sem_ra, sem_fa),
            (idx_b, rows_b, fov_b, sem_rb, sem_fb))

    def build_issue(p, buf):
        idx_v, rows_v, fov_v, sem_r, sem_f = buf
        p16 = jnp.full((L,), p, jnp.int32)
        offp = plsc.load_gather(offs_v, [p16])

        @plsc.parallel_loop(0, NB // L)
        def _(t):
            l16 = t * L + iota
            xi16 = plsc.load_gather(xi_c, [l16, p16])
            idx_v[pl.ds(t * L, L)] = xi16 + offp
        pltpu.async_copy(so_hbm.at[idx_v], rows_v, sem_r)
        pltpu.async_copy(fo_hbm.at[idx_v], fov_v, sem_f)

    def compute(p, buf):
        idx_v, rows_v, fov_v, sem_r, sem_f = buf
        pltpu.make_async_copy(so_hbm.at[idx_v], rows_v, sem_r).wait()
        pltpu.make_async_copy(fo_hbm.at[idx_v], fov_v, sem_f).wait()
        p16 = jnp.full((L,), p, jnp.int32)

        @plsc.parallel_loop(0, NB // L)
        def _(t):
            fo16 = fov_v[pl.ds(t * L, L)]
            xv16 = plsc.load_gather(xv_c, [t * L + iota, p16])
            plsc.addupdate(F_v.at[pl.ds(t * L, L)], fo16 * xv16)

        w0 = w_v[pl.ds(p * D + 0 * L, L)]
        w1 = w_v[pl.ds(p * D + 1 * L, L)]
        w2 = w_v[pl.ds(p * D + 2 * L, L)]
        w3 = w_v[pl.ds(p * D + 3 * L, L)]
        ws = (w0, w1, w2, w3)

        @plsc.parallel_loop(0, NB, unroll=4)
        def _(b):
            vb = plsc.load_gather(xv_c, [jnp.full((L,), b, jnp.int32), p16])
            cacc = zero16
            for d4 in range(D4):
                r = rows_v[b, pl.ds(d4 * L, L)]
                e = r * vb
                plsc.addupdate(S_v.at[pl.ds(b * D + d4 * L, L)], e)
                cacc = cacc + e * (ws[d4] - 0.5 * e)
            plsc.addupdate(C_v.at[pl.ds(b * L, L)], cacc)

    def chunk_body(c, _):
        base = wid * BPW + c * NB
        pltpu.sync_copy(xi_hbm.at[pl.ds(base, NB), :], xi_c)
        pltpu.sync_copy(xv_hbm.at[pl.ds(base, NB), :], xv_c)

        @plsc.parallel_loop(0, NB * D // L)
        def _(t):
            S_v[pl.ds(t * L, L)] = zero16

        @plsc.parallel_loop(0, NB * L // L)
        def _(t):
            C_v[pl.ds(t * L, L)] = zero16

        @plsc.parallel_loop(0, NB // L)
        def _(t):
            F_v[pl.ds(t * L, L)] = zero16

        def pair_body(k, _):
            build_issue(2 * k, bufs[0])
            build_issue(2 * k + 1, bufs[1])
            compute(2 * k, bufs[0])
            compute(2 * k + 1, bufs[1])
            return 0
        lax.fori_loop(0, NCOLS // 2, pair_body, 0)
        build_issue(NCOLS - 1, bufs[0])
        compute(NCOLS - 1, bufs[0])

        def fin_body(t, _):
            l16 = t * L + iota
            a = zero16
            for d in range(D):
                sd = plsc.load_gather(S_v, [l16 * D + d])
                a = a + sd * sd
            csum = zero16
            for k in range(L):
                ck = plsc.load_gather(C_v, [l16 * L + k])
                csum = csum + ck
            f16 = F_v[pl.ds(t * L, L)]
            out_v[pl.ds(t * L, L)] = f16 + 0.5 * a + csum + cst_v[...]
            return 0
        lax.fori_loop(0, NB // L, fin_body, 0)
        pltpu.sync_copy(out_v, out_hbm.at[pl.ds(base, NB)])
        return 0
    lax.fori_loop(0, NCHUNK, chunk_body, 0)


def kernel(Xi, Xv, fo_tables, so_tables, W1, b1, gamma1, beta1,
           W2, b2, gamma2, beta2, bias):
    a1 = 1.0 / jnp.sqrt(jnp.float32(1.0) + jnp.float32(BN_EPS))
    g2 = a1 * gamma2
    u = W2 @ g2
    g = a1 * gamma1 * u
    w_fold = W1 @ g
    cst = b1 @ g + beta1 @ u + b2 @ g2 + jnp.sum(beta2) + bias[0]
    cst_vec = jnp.full((L,), cst, jnp.float32)

    w_cols = w_fold.reshape(NCOLS, D)[np.argsort(_COLS)].reshape(NCOLS * D)
    offs = jnp.asarray(_OFFS_BY_COL)
    so_flat = so_tables.reshape(len(SECTIONS) * V, D)
    fo_flat = fo_tables.reshape(len(SECTIONS) * V)
    return _deepfm_sc(Xi, Xv, so_flat, fo_flat, w_cols, offs, cst_vec)
